# Initial kernel scaffold; baseline (speedup 1.0000x reference)
import jax
import jax.numpy as jnp
from jax.experimental import pallas as pl


def kernel(x, edge_index, batch, hyperparameters, params):
    raise NotImplementedError("write your pallas kernel here")


# speedup vs baseline: 5.2309x; 5.2309x over previous
import functools
import jax
import jax.numpy as jnp
from jax import lax
from jax.experimental import pallas as pl
from jax.experimental.pallas import tpu as pltpu
from jax.experimental.pallas import tpu_sc as plsc

N = 30000
E = 480000
NDIM = 64
NGRAPH = 1000
E2 = 2 * E
NC = 2
NS = 16
HALF = NDIM // 2
CHUNK = 120
INNER = 10
NPASS = 2
OUTER = E // (NS * CHUNK * INNER)
ACC_ROWS = 30720
ZSLICE = ACC_ROWS // NS

_mesh = plsc.VectorSubcoreMesh(core_axis_name="c", subcore_axis_name="s")


@functools.partial(
    pl.kernel,
    mesh=_mesh,
    compiler_params=pltpu.CompilerParams(use_tc_tiling_on_sc=False),
    out_type=jax.ShapeDtypeStruct((NC * NPASS * ACC_ROWS, HALF), jnp.float32),
    scratch_types=[
        pltpu.VMEM((2 * INNER, CHUNK), jnp.int32),
        pltpu.VMEM((INNER, CHUNK, HALF), jnp.float32),
        pltpu.VMEM_SHARED((ACC_ROWS, HALF), jnp.float32),
        pltpu.SemaphoreType.DMA,
    ],
)
def _edge_agg(idx_hbm, table_hbm, zeros_hbm, out_hbm, idx_v, rows_v, acc, sem):
    c = lax.axis_index("c")
    s = lax.axis_index("s")
    for p in range(NPASS):
        pltpu.sync_copy(zeros_hbm, acc.at[pl.ds(s * ZSLICE, ZSLICE)])
        plsc.subcore_barrier()

        def outer(j, carry):
            b = ((c * NPASS + p) * NS + s) * OUTER + j
            pltpu.sync_copy(idx_hbm.at[b], idx_v)
            for i in range(INNER):
                pltpu.async_copy(table_hbm.at[idx_v.at[2 * i]], rows_v.at[i], sem).wait()
                pltpu.sync_copy(rows_v.at[i], acc.at[idx_v.at[2 * i + 1]], add=True)
            return carry

        lax.fori_loop(0, OUTER, outer, 0)
        plsc.subcore_barrier()
        pltpu.sync_copy(acc.at[pl.ds(s * ZSLICE, ZSLICE)],
                        out_hbm.at[pl.ds((c * NPASS + p) * ACC_ROWS + s * ZSLICE,
                                         ZSLICE)])
        plsc.subcore_barrier()


def _message_pass(idx_pack, table, zeros):
    out = _edge_agg(idx_pack, table, zeros)
    out = out.reshape(NC, NPASS, ACC_ROWS, HALF)
    G1 = jnp.concatenate([out[0, 0, :N], out[1, 0, :N]], axis=1)
    G2 = jnp.concatenate([out[0, 1, :N], out[1, 1, :N]], axis=1)
    return G1, G2


def kernel(x, edge_index, batch, hyperparameters, params):
    return _run(x, edge_index, batch, hyperparameters, params)


def _run(x, edge_index, batch, hyperparameters, params):
    e0 = edge_index[0].astype(jnp.int32)
    e1 = edge_index[1].astype(jnp.int32)

    e0r = e0.reshape(NS, OUTER, INNER, 1, CHUNK)
    e1r = e1.reshape(NS, OUTER, INNER, 1, CHUNK)
    blocks = [jnp.concatenate([g + c * N, sct], axis=3)
              for c in range(NC) for (g, sct) in ((e0r, e1r), (e1r, e0r))]
    idx_pack = jnp.stack(blocks).reshape(NC * NPASS * NS * OUTER, 2 * INNER, CHUNK)

    zeros = jnp.zeros((ZSLICE, HALF), jnp.float32)

    indeg = jnp.zeros((N, 1), jnp.float32).at[e1].add(1.0)
    outdeg = jnp.zeros((N, 1), jnp.float32).at[e0].add(1.0)

    hps = hyperparameters.reshape(-1, 5)
    hps = jax.nn.relu(hps @ params['hyp_W'].T + params['hyp_b'])
    hyp_rep = jnp.repeat(hps, 30, axis=0)

    h = params['emb'][x]
    for l in range(2):
        W = params['msg_W%d' % l]
        Wr = params['msgr_W%d' % l]
        table = jnp.concatenate([h[:, :HALF], h[:, HALF:]], axis=0)
        G1, G2 = _message_pass(idx_pack, table, zeros)
        aggr = (G1 @ W[:, :NDIM].T + G2 @ Wr[:, :NDIM].T
                + indeg * (h @ W[:, NDIM:].T + params['msg_b%d' % l])
                + outdeg * (h @ Wr[:, NDIM:].T + params['msgr_b%d' % l]))
        gi = aggr @ params['Wih%d' % l].T + params['bih%d' % l]
        gh = h @ params['Whh%d' % l].T + params['bhh%d' % l]
        i_r, i_z, i_n = jnp.split(gi, 3, axis=1)
        h_r, h_z, h_n = jnp.split(gh, 3, axis=1)
        r = jax.nn.sigmoid(i_r + h_r)
        z = jax.nn.sigmoid(i_z + h_z)
        n = jnp.tanh(i_n + r * h_n)
        h = (1.0 - z) * n + z * h

    hcat = jnp.concatenate([h, hyp_rep], axis=1)
    h_vG = hcat @ params['fm_W'].T + params['fm_b']
    g = jax.nn.sigmoid(hcat @ params['gm_W'].T + params['gm_b'])
    h_G = jax.ops.segment_sum(h_vG * g, batch, num_segments=NGRAPH)

    hh = h_G
    for i in range(2):
        hh = jax.nn.relu(hh @ params['cls_W%d' % i].T + params['cls_b%d' % i])
    h_cls = jax.nn.softmax(hh @ params['cls_W2'].T + params['cls_b2'], axis=1)
    hc = jnp.concatenate([h_cls, h_G], axis=1)
    for i in range(2):
        hc = jax.nn.relu(hc @ params['reg_W%d' % i].T + params['reg_b%d' % i])
    hc = hc @ params['reg_W2'].T + params['reg_b2']
    return h_cls, hc.reshape(-1)
